# baseline (speedup 1.0000x reference)
import functools

import jax
import jax.numpy as jnp
from jax.experimental import pallas as pl
from jax.experimental.pallas import tpu as pltpu

BN_EPS = 1e-5
PW = 8


def _round_up(x, m):
    return (x + m - 1) // m * m


def _scratch_width(W):
    return PW + _round_up(W + 1, 8)


def _affine_from_stats(s_ref, ss_ref, g_ref, be_ref, cnt, total):
    C = s_ref.shape[-1]
    s = s_ref[:, 0, :]
    ss = ss_ref[:, 0, :]
    mean_p = s * (1.0 / cnt)
    m2_p = ss - s * mean_p
    mean = jnp.sum(s, axis=0, keepdims=True) * (1.0 / total)
    m2 = (jnp.sum(m2_p, axis=0, keepdims=True)
          + cnt * jnp.sum((mean_p - mean) ** 2, axis=0, keepdims=True))
    var = m2 * (1.0 / total)
    scale = g_ref[...].reshape(1, C) * jax.lax.rsqrt(var + BN_EPS)
    shift = be_ref[...].reshape(1, C) - mean * scale
    return scale, shift


def _conv_stage_kernel(x_ref, sp_ref, ssp_ref, g_ref, be_ref, w_ref, b_ref,
                       y_ref, s_ref, ss_ref, scr_ref,
                       *, nchw_in, hw, tr, cnt, total):
    H, W = hw
    Ci = w_ref.shape[1] // 3
    Co = w_ref.shape[-1]
    Wp = scr_ref.shape[1]

    scr_ref[:, 0:PW, :] = jnp.zeros((H + 2, PW, Ci), jnp.bfloat16)
    scr_ref[:, PW + W:, :] = jnp.zeros((H + 2, Wp - PW - W, Ci), jnp.bfloat16)
    scr_ref[0:1, PW:PW + W, :] = jnp.zeros((1, W, Ci), jnp.bfloat16)
    scr_ref[H + 1:H + 2, PW:PW + W, :] = jnp.zeros((1, W, Ci), jnp.bfloat16)

    if nchw_in:
        xb = jnp.transpose(x_ref[0].astype(jnp.bfloat16), (1, 0))
        xb = xb.reshape(H, W, Ci)
    else:
        sc, sh = _affine_from_stats(sp_ref, ssp_ref, g_ref, be_ref, cnt, total)
        xb = x_ref[0].astype(jnp.float32) * sc.reshape(1, 1, Ci)
        xb = jnp.maximum(xb + sh.reshape(1, 1, Ci), 0.0).astype(jnp.bfloat16)
    scr_ref[1:H + 1, PW:PW + W, :] = xb

    bias = b_ref[...]
    s_tot = jnp.zeros((1, Co), jnp.float32)
    ss_tot = jnp.zeros((1, Co), jnp.float32)
    for r0 in range(0, H, tr):
        acc = jnp.zeros((tr * W, Co), jnp.float32)
        for dx in range(3):
            c0 = PW - 1 + dx
            lhs = jnp.concatenate(
                [scr_ref[r0 + dy:r0 + dy + tr, c0:c0 + W, :]
                 for dy in range(3)], axis=-1).reshape(tr * W, 3 * Ci)
            acc += jnp.dot(lhs, w_ref[dx],
                           preferred_element_type=jnp.float32)
        acc += bias
        y_ref[0, r0:r0 + tr, :, :] = acc.reshape(tr, W, Co).astype(jnp.bfloat16)
        s_tot = s_tot + jnp.sum(acc, axis=0, keepdims=True)
        ss_tot = ss_tot + jnp.sum(acc * acc, axis=0, keepdims=True)

    s_ref[...] = jnp.broadcast_to(s_tot.reshape(1, 1, Co), (1, 8, Co))
    ss_ref[...] = jnp.broadcast_to(ss_tot.reshape(1, 1, Co), (1, 8, Co))


def _conv_stage(x, s_prev, ss_prev, g, be, w_stacked, b, *, nchw_in, hw, tr):
    H, W = hw
    N = x.shape[0]
    Ci = w_stacked.shape[1] // 3
    Co = w_stacked.shape[-1]
    wp = _scratch_width(W)
    cnt = float(H * W)
    total = float(N * H * W)

    x_spec = (pl.BlockSpec((1, Ci, H * W), lambda n: (n, 0, 0)) if nchw_in
              else pl.BlockSpec((1, H, W, Ci), lambda n: (n, 0, 0, 0)))
    body = functools.partial(_conv_stage_kernel, nchw_in=nchw_in, hw=hw,
                             tr=tr, cnt=cnt, total=total)
    return pl.pallas_call(
        body,
        grid=(N,),
        in_specs=[
            x_spec,
            pl.BlockSpec((N, 8, Ci), lambda n: (0, 0, 0)),
            pl.BlockSpec((N, 8, Ci), lambda n: (0, 0, 0)),
            pl.BlockSpec((1, Ci), lambda n: (0, 0)),
            pl.BlockSpec((1, Ci), lambda n: (0, 0)),
            pl.BlockSpec((3, 3 * Ci, Co), lambda n: (0, 0, 0)),
            pl.BlockSpec((1, Co), lambda n: (0, 0)),
        ],
        out_specs=(
            pl.BlockSpec((1, H, W, Co), lambda n: (n, 0, 0, 0)),
            pl.BlockSpec((1, 8, Co), lambda n: (n, 0, 0)),
            pl.BlockSpec((1, 8, Co), lambda n: (n, 0, 0)),
        ),
        out_shape=(
            jax.ShapeDtypeStruct((N, H, W, Co), jnp.bfloat16),
            jax.ShapeDtypeStruct((N, 8, Co), jnp.float32),
            jax.ShapeDtypeStruct((N, 8, Co), jnp.float32),
        ),
        scratch_shapes=[
            pltpu.VMEM((H + 2, wp, Ci), jnp.bfloat16),
        ],
        compiler_params=pltpu.CompilerParams(
            dimension_semantics=("parallel",),
            vmem_limit_bytes=48 * 1024 * 1024),
    )(x, s_prev, ss_prev, g, be, w_stacked, b)


def _norm_relu_kernel(y_ref, sp_ref, ssp_ref, g_ref, be_ref, o_ref,
                      *, cnt, total):
    HW, C = o_ref.shape[2], o_ref.shape[1]
    sc, sh = _affine_from_stats(sp_ref, ssp_ref, g_ref, be_ref, cnt, total)
    v = y_ref[0].reshape(HW, C).astype(jnp.float32) * sc
    v = jnp.maximum(v + sh, 0.0)
    o_ref[0] = jnp.transpose(v, (1, 0))


def _norm_relu(y, s_prev, ss_prev, g, be):
    N, H, W, C = y.shape
    cnt = float(H * W)
    total = float(N * H * W)
    body = functools.partial(_norm_relu_kernel, cnt=cnt, total=total)
    return pl.pallas_call(
        body,
        grid=(N,),
        in_specs=[
            pl.BlockSpec((1, H, W, C), lambda n: (n, 0, 0, 0)),
            pl.BlockSpec((N, 8, C), lambda n: (0, 0, 0)),
            pl.BlockSpec((N, 8, C), lambda n: (0, 0, 0)),
            pl.BlockSpec((1, C), lambda n: (0, 0)),
            pl.BlockSpec((1, C), lambda n: (0, 0)),
        ],
        out_specs=pl.BlockSpec((1, C, H * W), lambda n: (n, 0, 0)),
        out_shape=jax.ShapeDtypeStruct((N, C, H * W), jnp.float32),
        compiler_params=pltpu.CompilerParams(
            dimension_semantics=("parallel",),
            vmem_limit_bytes=48 * 1024 * 1024),
    )(y, s_prev, ss_prev, g, be)


def _stack_dy(w):
    return jnp.stack(
        [jnp.concatenate([w[dy, dx] for dy in range(3)], axis=0)
         for dx in range(3)], axis=0).astype(jnp.bfloat16)


def kernel(x, w1, b1, g1, be1, w2, b2, g2, be2):
    N, Cin, H, W = x.shape
    Cout = w1.shape[-1]
    tr = 4 if (H % 4 == 0) else 1

    x_flat = x.reshape(N, Cin, H * W)
    w1s = _stack_dy(w1)
    w2s = _stack_dy(w2)
    b1r = b1.reshape(1, Cout).astype(jnp.float32)
    b2r = b2.reshape(1, Cout).astype(jnp.float32)
    zs = jnp.zeros((N, 8, Cin), jnp.float32)
    za = jnp.zeros((1, Cin), jnp.float32)

    y1, s1, ss1 = _conv_stage(x_flat, zs, zs, za, za, w1s, b1r,
                              nchw_in=True, hw=(H, W), tr=tr)
    y2, s2, ss2 = _conv_stage(y1, s1, ss1, g1.astype(jnp.float32),
                              be1.astype(jnp.float32), w2s, b2r,
                              nchw_in=False, hw=(H, W), tr=tr)
    out = _norm_relu(y2, s2, ss2, g2.astype(jnp.float32),
                     be2.astype(jnp.float32))
    return out.reshape(N, Cout, H, W)


# speedup vs baseline: 1.0667x; 1.0667x over previous
import functools

import jax
import jax.numpy as jnp
from jax.experimental import pallas as pl
from jax.experimental.pallas import tpu as pltpu

BN_EPS = 1e-5
PW = 8


def _round_up(x, m):
    return (x + m - 1) // m * m


def _scratch_width(W):
    return PW + _round_up(W + 1, 8)


def _affine_from_stats(s_ref, ss_ref, g_ref, be_ref, cnt, total):
    C = s_ref.shape[-1]
    s = s_ref[:, 0, :]
    ss = ss_ref[:, 0, :]
    mean_p = s * (1.0 / cnt)
    m2_p = ss - s * mean_p
    mean = jnp.sum(s, axis=0, keepdims=True) * (1.0 / total)
    m2 = (jnp.sum(m2_p, axis=0, keepdims=True)
          + cnt * jnp.sum((mean_p - mean) ** 2, axis=0, keepdims=True))
    var = m2 * (1.0 / total)
    scale = g_ref[...].reshape(1, C) * jax.lax.rsqrt(var + BN_EPS)
    shift = be_ref[...].reshape(1, C) - mean * scale
    return scale, shift


def _conv_stage_kernel(x_ref, sp_ref, ssp_ref, g_ref, be_ref, w_ref, b_ref,
                       y_ref, s_ref, ss_ref, scr_ref,
                       *, nchw_in, hw, tr, cnt, total, raw_in=False):
    H, W = hw
    Ci = w_ref.shape[1] // 3
    Co = w_ref.shape[-1]
    Wp = scr_ref.shape[1]

    scr_ref[:, 0:PW, :] = jnp.zeros((H + 2, PW, Ci), jnp.bfloat16)
    scr_ref[:, PW + W:, :] = jnp.zeros((H + 2, Wp - PW - W, Ci), jnp.bfloat16)
    scr_ref[0:1, PW:PW + W, :] = jnp.zeros((1, W, Ci), jnp.bfloat16)
    scr_ref[H + 1:H + 2, PW:PW + W, :] = jnp.zeros((1, W, Ci), jnp.bfloat16)

    if nchw_in:
        xb = jnp.transpose(x_ref[0].astype(jnp.bfloat16), (1, 0))
        xb = xb.reshape(H, W, Ci)
    elif raw_in:
        xb = x_ref[0]
    else:
        sc, sh = _affine_from_stats(sp_ref, ssp_ref, g_ref, be_ref, cnt, total)
        xb = x_ref[0].astype(jnp.float32) * sc.reshape(1, 1, Ci)
        xb = jnp.maximum(xb + sh.reshape(1, 1, Ci), 0.0).astype(jnp.bfloat16)
    scr_ref[1:H + 1, PW:PW + W, :] = xb

    bias = b_ref[...]
    s_tot = jnp.zeros((1, Co), jnp.float32)
    ss_tot = jnp.zeros((1, Co), jnp.float32)
    for r0 in range(0, H, tr):
        acc = jnp.zeros((tr * W, Co), jnp.float32)
        for dx in range(3):
            c0 = PW - 1 + dx
            lhs = jnp.concatenate(
                [scr_ref[r0 + dy:r0 + dy + tr, c0:c0 + W, :]
                 for dy in range(3)], axis=-1).reshape(tr * W, 3 * Ci)
            acc += jnp.dot(lhs, w_ref[dx],
                           preferred_element_type=jnp.float32)
        acc += bias
        y_ref[0, r0:r0 + tr, :, :] = acc.reshape(tr, W, Co).astype(jnp.bfloat16)
        s_tot = s_tot + jnp.sum(acc, axis=0, keepdims=True)
        ss_tot = ss_tot + jnp.sum(acc * acc, axis=0, keepdims=True)

    s_ref[...] = jnp.broadcast_to(s_tot.reshape(1, 1, Co), (1, 8, Co))
    ss_ref[...] = jnp.broadcast_to(ss_tot.reshape(1, 1, Co), (1, 8, Co))


def _conv_stage(x, s_prev, ss_prev, g, be, w_stacked, b, *, nchw_in, hw, tr,
                raw_in=False):
    H, W = hw
    N = x.shape[0]
    Ci = w_stacked.shape[1] // 3
    Co = w_stacked.shape[-1]
    wp = _scratch_width(W)
    cnt = float(H * W)
    total = float(N * H * W)

    x_spec = (pl.BlockSpec((1, Ci, H * W), lambda n: (n, 0, 0)) if nchw_in
              else pl.BlockSpec((1, H, W, Ci), lambda n: (n, 0, 0, 0)))
    body = functools.partial(_conv_stage_kernel, nchw_in=nchw_in, hw=hw,
                             tr=tr, cnt=cnt, total=total, raw_in=raw_in)
    return pl.pallas_call(
        body,
        grid=(N,),
        in_specs=[
            x_spec,
            pl.BlockSpec((N, 8, Ci), lambda n: (0, 0, 0)),
            pl.BlockSpec((N, 8, Ci), lambda n: (0, 0, 0)),
            pl.BlockSpec((1, Ci), lambda n: (0, 0)),
            pl.BlockSpec((1, Ci), lambda n: (0, 0)),
            pl.BlockSpec((3, 3 * Ci, Co), lambda n: (0, 0, 0)),
            pl.BlockSpec((1, Co), lambda n: (0, 0)),
        ],
        out_specs=(
            pl.BlockSpec((1, H, W, Co), lambda n: (n, 0, 0, 0)),
            pl.BlockSpec((1, 8, Co), lambda n: (n, 0, 0)),
            pl.BlockSpec((1, 8, Co), lambda n: (n, 0, 0)),
        ),
        out_shape=(
            jax.ShapeDtypeStruct((N, H, W, Co), jnp.bfloat16),
            jax.ShapeDtypeStruct((N, 8, Co), jnp.float32),
            jax.ShapeDtypeStruct((N, 8, Co), jnp.float32),
        ),
        scratch_shapes=[
            pltpu.VMEM((H + 2, wp, Ci), jnp.bfloat16),
        ],
        compiler_params=pltpu.CompilerParams(
            dimension_semantics=("parallel",),
            vmem_limit_bytes=48 * 1024 * 1024),
    )(x, s_prev, ss_prev, g, be, w_stacked, b)


def _norm_relu_kernel(y_ref, sp_ref, ssp_ref, g_ref, be_ref, o_ref,
                      *, cnt, total):
    HW, C = o_ref.shape[2], o_ref.shape[1]
    sc, sh = _affine_from_stats(sp_ref, ssp_ref, g_ref, be_ref, cnt, total)
    v = y_ref[0].reshape(HW, C).astype(jnp.float32) * sc
    v = jnp.maximum(v + sh, 0.0)
    o_ref[0] = jnp.transpose(v, (1, 0))


def _norm_relu(y, s_prev, ss_prev, g, be):
    N, H, W, C = y.shape
    cnt = float(H * W)
    total = float(N * H * W)
    body = functools.partial(_norm_relu_kernel, cnt=cnt, total=total)
    return pl.pallas_call(
        body,
        grid=(N,),
        in_specs=[
            pl.BlockSpec((1, H, W, C), lambda n: (n, 0, 0, 0)),
            pl.BlockSpec((N, 8, C), lambda n: (0, 0, 0)),
            pl.BlockSpec((N, 8, C), lambda n: (0, 0, 0)),
            pl.BlockSpec((1, C), lambda n: (0, 0)),
            pl.BlockSpec((1, C), lambda n: (0, 0)),
        ],
        out_specs=pl.BlockSpec((1, C, H * W), lambda n: (n, 0, 0)),
        out_shape=jax.ShapeDtypeStruct((N, C, H * W), jnp.float32),
        compiler_params=pltpu.CompilerParams(
            dimension_semantics=("parallel",),
            vmem_limit_bytes=48 * 1024 * 1024),
    )(y, s_prev, ss_prev, g, be)


def _stack_dy(w):
    return jnp.stack(
        [jnp.concatenate([w[dy, dx] for dy in range(3)], axis=0)
         for dx in range(3)], axis=0).astype(jnp.bfloat16)


def kernel(x, w1, b1, g1, be1, w2, b2, g2, be2):
    N, Cin, H, W = x.shape
    Cout = w1.shape[-1]
    tr = 4 if (H % 4 == 0) else 1

    x_nhwc = jnp.transpose(x, (0, 2, 3, 1)).astype(jnp.bfloat16)
    w1s = _stack_dy(w1)
    w2s = _stack_dy(w2)
    b1r = b1.reshape(1, Cout).astype(jnp.float32)
    b2r = b2.reshape(1, Cout).astype(jnp.float32)
    zs = jnp.zeros((N, 8, Cin), jnp.float32)
    za = jnp.zeros((1, Cin), jnp.float32)

    y1, s1, ss1 = _conv_stage(x_nhwc, zs, zs, za, za, w1s, b1r,
                              nchw_in=False, hw=(H, W), tr=tr, raw_in=True)
    y2, s2, ss2 = _conv_stage(y1, s1, ss1, g1.astype(jnp.float32),
                              be1.astype(jnp.float32), w2s, b2r,
                              nchw_in=False, hw=(H, W), tr=tr)
    out = _norm_relu(y2, s2, ss2, g2.astype(jnp.float32),
                     be2.astype(jnp.float32))
    return out.reshape(N, Cout, H, W)


# speedup vs baseline: 1.0699x; 1.0030x over previous
import functools

import jax
import jax.numpy as jnp
from jax.experimental import pallas as pl
from jax.experimental.pallas import tpu as pltpu

BN_EPS = 1e-5
PW = 8


def _round_up(x, m):
    return (x + m - 1) // m * m


def _scratch_width(W):
    return PW + _round_up(W + 1, 8)


def _conv_stage_kernel(xb_ref, scale_ref, shift_ref, w_ref, b_ref,
                       y_ref, s_ref, ss_ref, scr_ref, *, act_input, tr):
    _, H, W, Ci = xb_ref.shape
    Co = w_ref.shape[-1]
    Wp = scr_ref.shape[1]

    scr_ref[:, 0:PW, :] = jnp.zeros((H + 2, PW, Ci), jnp.bfloat16)
    scr_ref[:, PW + W:, :] = jnp.zeros((H + 2, Wp - PW - W, Ci), jnp.bfloat16)
    scr_ref[0:1, PW:PW + W, :] = jnp.zeros((1, W, Ci), jnp.bfloat16)
    scr_ref[H + 1:H + 2, PW:PW + W, :] = jnp.zeros((1, W, Ci), jnp.bfloat16)

    xb = xb_ref[0]
    if act_input:
        sc = scale_ref[...].reshape(1, 1, Ci)
        sh = shift_ref[...].reshape(1, 1, Ci)
        xb = jnp.maximum(xb.astype(jnp.float32) * sc + sh, 0.0)
    scr_ref[1:H + 1, PW:PW + W, :] = xb.astype(jnp.bfloat16)

    bias = b_ref[...]
    s_tot = jnp.zeros((1, Co), jnp.float32)
    ss_tot = jnp.zeros((1, Co), jnp.float32)
    for r0 in range(0, H, tr):
        acc = jnp.zeros((tr * W, Co), jnp.float32)
        for dx in range(3):
            c0 = PW - 1 + dx
            lhs = jnp.concatenate(
                [scr_ref[r0 + dy:r0 + dy + tr, c0:c0 + W, :]
                 for dy in range(3)], axis=-1).reshape(tr * W, 3 * Ci)
            acc += jnp.dot(lhs, w_ref[dx],
                           preferred_element_type=jnp.float32)
        acc += bias
        y_ref[0, r0:r0 + tr, :, :] = acc.reshape(tr, W, Co).astype(jnp.bfloat16)
        s_tot = s_tot + jnp.sum(acc, axis=0, keepdims=True)
        ss_tot = ss_tot + jnp.sum(acc * acc, axis=0, keepdims=True)

    s_ref[...] = jnp.broadcast_to(s_tot.reshape(1, 1, Co), (1, 8, Co))
    ss_ref[...] = jnp.broadcast_to(ss_tot.reshape(1, 1, Co), (1, 8, Co))


def _conv_stage(x, scale, shift, w_stacked, b, *, act_input, tr):
    N, H, W, Ci = x.shape
    Co = w_stacked.shape[-1]
    wp = _scratch_width(W)

    body = functools.partial(_conv_stage_kernel, act_input=act_input, tr=tr)
    return pl.pallas_call(
        body,
        grid=(N,),
        in_specs=[
            pl.BlockSpec((1, H, W, Ci), lambda n: (n, 0, 0, 0)),
            pl.BlockSpec((1, Ci), lambda n: (0, 0)),
            pl.BlockSpec((1, Ci), lambda n: (0, 0)),
            pl.BlockSpec((3, 3 * Ci, Co), lambda n: (0, 0, 0)),
            pl.BlockSpec((1, Co), lambda n: (0, 0)),
        ],
        out_specs=(
            pl.BlockSpec((1, H, W, Co), lambda n: (n, 0, 0, 0)),
            pl.BlockSpec((1, 8, Co), lambda n: (n, 0, 0)),
            pl.BlockSpec((1, 8, Co), lambda n: (n, 0, 0)),
        ),
        out_shape=(
            jax.ShapeDtypeStruct((N, H, W, Co), jnp.bfloat16),
            jax.ShapeDtypeStruct((N, 8, Co), jnp.float32),
            jax.ShapeDtypeStruct((N, 8, Co), jnp.float32),
        ),
        scratch_shapes=[
            pltpu.VMEM((H + 2, wp, Ci), jnp.bfloat16),
        ],
        compiler_params=pltpu.CompilerParams(
            dimension_semantics=("parallel",),
            vmem_limit_bytes=48 * 1024 * 1024),
    )(x, scale, shift, w_stacked, b)


def _norm_relu_kernel(y_ref, scale_ref, shift_ref, o_ref):
    HW, C = o_ref.shape[2], o_ref.shape[1]
    sc = scale_ref[...]
    sh = shift_ref[...]
    v = y_ref[0].reshape(HW, C).astype(jnp.float32) * sc
    v = jnp.maximum(v + sh, 0.0)
    o_ref[0] = jnp.transpose(v, (1, 0))


def _norm_relu(y, scale, shift):
    N, H, W, C = y.shape
    return pl.pallas_call(
        _norm_relu_kernel,
        grid=(N,),
        in_specs=[
            pl.BlockSpec((1, H, W, C), lambda n: (n, 0, 0, 0)),
            pl.BlockSpec((1, C), lambda n: (0, 0)),
            pl.BlockSpec((1, C), lambda n: (0, 0)),
        ],
        out_specs=pl.BlockSpec((1, C, H * W), lambda n: (n, 0, 0)),
        out_shape=jax.ShapeDtypeStruct((N, C, H * W), jnp.float32),
        compiler_params=pltpu.CompilerParams(
            dimension_semantics=("parallel",),
            vmem_limit_bytes=48 * 1024 * 1024),
    )(y, scale, shift)


def _bn_affine(s_part, ss_part, gamma, beta, cnt, total):
    C = s_part.shape[-1]
    s = s_part.reshape(-1, C)
    ss = ss_part.reshape(-1, C)
    mean_p = s / cnt
    m2_p = ss - s * mean_p
    mean = jnp.sum(s, axis=0) / total
    m2 = jnp.sum(m2_p, axis=0) + cnt * jnp.sum((mean_p - mean) ** 2, axis=0)
    var = m2 / total
    scale = gamma.reshape(-1) * jax.lax.rsqrt(var + BN_EPS)
    shift = beta.reshape(-1) - mean * scale
    return scale.reshape(1, C), shift.reshape(1, C)


def _stack_dy(w):
    return jnp.stack(
        [jnp.concatenate([w[dy, dx] for dy in range(3)], axis=0)
         for dx in range(3)], axis=0).astype(jnp.bfloat16)


def kernel(x, w1, b1, g1, be1, w2, b2, g2, be2):
    N, Cin, H, W = x.shape
    Cout = w1.shape[-1]
    tr = 4 if (H % 4 == 0) else 1

    xh = jnp.transpose(x, (0, 2, 3, 1)).astype(jnp.bfloat16)

    w1s = _stack_dy(w1)
    w2s = _stack_dy(w2)
    b1r = b1.reshape(1, Cout).astype(jnp.float32)
    b2r = b2.reshape(1, Cout).astype(jnp.float32)
    no_aff = jnp.zeros((1, Cin), jnp.float32)

    cnt = float(H * W)
    total = float(N * H * W)

    y1, s1, ss1 = _conv_stage(xh, no_aff, no_aff, w1s, b1r,
                              act_input=False, tr=tr)
    sc1, sh1 = _bn_affine(s1[:, 0, :], ss1[:, 0, :], g1, be1, cnt, total)

    y2, s2, ss2 = _conv_stage(y1, sc1, sh1, w2s, b2r,
                              act_input=True, tr=tr)
    sc2, sh2 = _bn_affine(s2[:, 0, :], ss2[:, 0, :], g2, be2, cnt, total)

    out = _norm_relu(y2, sc2, sh2)
    return out.reshape(N, Cout, H, W)


# speedup vs baseline: 1.5241x; 1.4245x over previous
import functools

import jax
import jax.numpy as jnp
from jax.experimental import pallas as pl
from jax.experimental.pallas import tpu as pltpu

BN_EPS = 1e-5


def _conv_stage_kernel(xb_ref, scale_ref, shift_ref, w_ref, b_ref,
                       y_ref, s_ref, ss_ref, scr_ref, *, act_input, tr):
    _, H, W, Ci = xb_ref.shape
    Co = w_ref.shape[-1] // 3

    xb = xb_ref[0]
    if act_input:
        sc = scale_ref[...].reshape(1, 1, Ci)
        sh = shift_ref[...].reshape(1, 1, Ci)
        xb = jnp.maximum(xb.astype(jnp.float32) * sc + sh, 0.0)
    xb = xb.astype(jnp.bfloat16)
    scr_ref[:, :, Ci:2 * Ci] = xb
    scr_ref[1:H, :, 0:Ci] = xb[0:H - 1]
    scr_ref[0:1, :, 0:Ci] = jnp.zeros((1, W, Ci), jnp.bfloat16)
    scr_ref[0:H - 1, :, 2 * Ci:3 * Ci] = xb[1:H]
    scr_ref[H - 1:H, :, 2 * Ci:3 * Ci] = jnp.zeros((1, W, Ci), jnp.bfloat16)

    bias = b_ref[...]
    M = tr * W
    iota = jax.lax.broadcasted_iota(jnp.int32, (M, 1), 0)
    mask_l = (iota % W != 0).astype(jnp.float32)
    mask_r = (iota % W != W - 1).astype(jnp.float32)
    s_tot = jnp.zeros((1, Co), jnp.float32)
    ss_tot = jnp.zeros((1, Co), jnp.float32)
    for r0 in range(0, H, tr):
        lhs = scr_ref[r0:r0 + tr].reshape(M, 3 * Ci)
        pair = jnp.dot(lhs, w_ref[:, 0:2 * Co],
                       preferred_element_type=jnp.float32)
        acc = jnp.dot(lhs, w_ref[:, 2 * Co:3 * Co],
                      preferred_element_type=jnp.float32)
        acc = acc + jnp.roll(pair[:, 0:Co], 1, axis=0) * mask_l
        acc = acc + jnp.roll(pair[:, Co:2 * Co], -1, axis=0) * mask_r
        acc = acc + bias
        y_ref[0, r0:r0 + tr, :, :] = acc.reshape(tr, W, Co).astype(jnp.bfloat16)
        s_tot = s_tot + jnp.sum(acc, axis=0, keepdims=True)
        ss_tot = ss_tot + jnp.sum(acc * acc, axis=0, keepdims=True)

    s_ref[...] = jnp.broadcast_to(s_tot.reshape(1, 1, Co), (1, 8, Co))
    ss_ref[...] = jnp.broadcast_to(ss_tot.reshape(1, 1, Co), (1, 8, Co))


def _conv_stage(x, scale, shift, w_packed, b, *, act_input, tr):
    N, H, W, Ci = x.shape
    Co = w_packed.shape[-1] // 3

    body = functools.partial(_conv_stage_kernel, act_input=act_input, tr=tr)
    return pl.pallas_call(
        body,
        grid=(N,),
        in_specs=[
            pl.BlockSpec((1, H, W, Ci), lambda n: (n, 0, 0, 0)),
            pl.BlockSpec((1, Ci), lambda n: (0, 0)),
            pl.BlockSpec((1, Ci), lambda n: (0, 0)),
            pl.BlockSpec((3 * Ci, 3 * Co), lambda n: (0, 0)),
            pl.BlockSpec((1, Co), lambda n: (0, 0)),
        ],
        out_specs=(
            pl.BlockSpec((1, H, W, Co), lambda n: (n, 0, 0, 0)),
            pl.BlockSpec((1, 8, Co), lambda n: (n, 0, 0)),
            pl.BlockSpec((1, 8, Co), lambda n: (n, 0, 0)),
        ),
        out_shape=(
            jax.ShapeDtypeStruct((N, H, W, Co), jnp.bfloat16),
            jax.ShapeDtypeStruct((N, 8, Co), jnp.float32),
            jax.ShapeDtypeStruct((N, 8, Co), jnp.float32),
        ),
        scratch_shapes=[
            pltpu.VMEM((H, W, 3 * Ci), jnp.bfloat16),
        ],
        compiler_params=pltpu.CompilerParams(
            dimension_semantics=("parallel",),
            vmem_limit_bytes=48 * 1024 * 1024),
    )(x, scale, shift, w_packed, b)


def _norm_relu_kernel(y_ref, scale_ref, shift_ref, o_ref):
    C = y_ref.shape[-1]
    sc = scale_ref[...].reshape(1, 1, 1, C)
    sh = shift_ref[...].reshape(1, 1, 1, C)
    v = jnp.maximum(y_ref[...].astype(jnp.float32) * sc + sh, 0.0)
    o_ref[...] = v.astype(jnp.bfloat16)


def _norm_relu(y, scale, shift):
    N, H, W, C = y.shape
    return pl.pallas_call(
        _norm_relu_kernel,
        grid=(N,),
        in_specs=[
            pl.BlockSpec((1, H, W, C), lambda n: (n, 0, 0, 0)),
            pl.BlockSpec((1, C), lambda n: (0, 0)),
            pl.BlockSpec((1, C), lambda n: (0, 0)),
        ],
        out_specs=pl.BlockSpec((1, H, W, C), lambda n: (n, 0, 0, 0)),
        out_shape=jax.ShapeDtypeStruct((N, H, W, C), jnp.bfloat16),
        compiler_params=pltpu.CompilerParams(
            dimension_semantics=("parallel",),
            vmem_limit_bytes=32 * 1024 * 1024),
    )(y, scale, shift)


def _bn_affine(s_part, ss_part, gamma, beta, cnt, total):
    C = s_part.shape[-1]
    s = s_part.reshape(-1, C)
    ss = ss_part.reshape(-1, C)
    mean_p = s / cnt
    m2_p = ss - s * mean_p
    mean = jnp.sum(s, axis=0) / total
    m2 = jnp.sum(m2_p, axis=0) + cnt * jnp.sum((mean_p - mean) ** 2, axis=0)
    var = m2 / total
    scale = gamma.reshape(-1) * jax.lax.rsqrt(var + BN_EPS)
    shift = beta.reshape(-1) - mean * scale
    return scale.reshape(1, C), shift.reshape(1, C)


def _pack_w(w):
    slabs = [jnp.concatenate([w[dy, dx] for dy in range(3)], axis=0)
             for dx in range(3)]
    return jnp.concatenate([slabs[0], slabs[2], slabs[1]],
                           axis=1).astype(jnp.bfloat16)


def kernel(x, w1, b1, g1, be1, w2, b2, g2, be2):
    N, Cin, H, W = x.shape
    Cout = w1.shape[-1]
    tr = 4 if (H % 4 == 0) else 1

    xh = jnp.transpose(x, (0, 2, 3, 1)).astype(jnp.bfloat16)

    w1p = _pack_w(w1)
    w2p = _pack_w(w2)
    b1r = b1.reshape(1, Cout).astype(jnp.float32)
    b2r = b2.reshape(1, Cout).astype(jnp.float32)
    no_aff = jnp.zeros((1, Cin), jnp.float32)

    cnt = float(H * W)
    total = float(N * H * W)

    y1, s1, ss1 = _conv_stage(xh, no_aff, no_aff, w1p, b1r,
                              act_input=False, tr=tr)
    sc1, sh1 = _bn_affine(s1[:, 0, :], ss1[:, 0, :], g1, be1, cnt, total)

    y2, s2, ss2 = _conv_stage(y1, sc1, sh1, w2p, b2r,
                              act_input=True, tr=tr)
    sc2, sh2 = _bn_affine(s2[:, 0, :], ss2[:, 0, :], g2, be2, cnt, total)

    out = _norm_relu(y2, sc2, sh2)
    return jnp.transpose(out, (0, 3, 1, 2)).astype(jnp.float32)
